# baseline (speedup 1.0000x reference)
import functools

import jax
import jax.numpy as jnp
from jax import lax
from jax.experimental import pallas as pl
from jax.experimental.pallas import tpu as pltpu
from jax.experimental.pallas import tpu_sc as plsc


def _make_sc_broadcast(B, L, D, NC, NS):
    NW = NC * NS
    rows_per_w = B // NW
    row_words = L * D
    rep = 1
    for cand in range(min(rows_per_w, (120 * 1024) // row_words), 0, -1):
        if rows_per_w % cand == 0 and cand * row_words * 4 <= 480 * 1024:
            rep = cand
            break
    n_dma = rows_per_w // rep

    mesh = plsc.VectorSubcoreMesh(core_axis_name="c", subcore_axis_name="s")

    @functools.partial(
        pl.kernel,
        mesh=mesh,
        out_type=jax.ShapeDtypeStruct((B, row_words), jnp.float32),
        scratch_types=[
            pltpu.VMEM((rep, row_words), jnp.float32),
            pltpu.SemaphoreType.DMA,
        ],
    )
    def k(table_hbm, out_hbm, buf, sem):
        wid = lax.axis_index("s") * NC + lax.axis_index("c")
        pltpu.sync_copy(table_hbm, buf.at[0])

        def replicate(i, carry):
            v = buf[0, pl.ds(i * 16, 16)]
            for r in range(1, rep):
                buf[r, pl.ds(i * 16, 16)] = v
            return carry

        lax.fori_loop(0, row_words // 16, replicate, 0)
        copies = [
            pltpu.async_copy(
                buf, out_hbm.at[pl.ds((i * NW + wid) * rep, rep)], sem
            )
            for i in range(n_dma)
        ]
        for c in copies:
            c.wait()

    return k


def _tc_relayout(compact, B, L, D, bb):
    def body(i_ref, o_ref):
        o_ref[...] = i_ref[...].reshape(o_ref.shape)

    return pl.pallas_call(
        body,
        grid=(B // bb,),
        in_specs=[pl.BlockSpec((bb, L * D), lambda i: (i, 0))],
        out_specs=pl.BlockSpec((bb, L, D), lambda i: (i, 0, 0)),
        out_shape=jax.ShapeDtypeStruct((B, L, D), jnp.float32),
    )(compact)


def kernel(x, pos_table):
    B, L = x.shape
    D = pos_table.shape[1]
    info = plsc.get_sparse_core_info()
    NC, NS = info.num_cores, info.num_subcores
    table_flat = pos_table[:L].reshape(L * D)
    compact = _make_sc_broadcast(B, L, D, NC, NS)(table_flat)
    return _tc_relayout(compact, B, L, D, bb=128)


# speedup vs baseline: 1.2758x; 1.2758x over previous
import functools

import jax
import jax.numpy as jnp
from jax import lax
from jax.experimental import pallas as pl
from jax.experimental.pallas import tpu as pltpu
from jax.experimental.pallas import tpu_sc as plsc


def _make_sc_broadcast(B, L, D, NC, NS):
    NW = NC * NS
    rows_per_w = B // NW
    padded_row = L * max(D, 128) * 4
    rep = 1
    for cand in range(min(rows_per_w, (448 * 1024) // padded_row), 0, -1):
        if rows_per_w % cand == 0:
            rep = cand
            break
    n_dma = rows_per_w // rep

    mesh = plsc.VectorSubcoreMesh(core_axis_name="c", subcore_axis_name="s")

    @functools.partial(
        pl.kernel,
        mesh=mesh,
        out_type=jax.ShapeDtypeStruct((B, L, D), jnp.float32),
        scratch_types=[
            pltpu.VMEM((rep, L, D), jnp.float32),
            pltpu.SemaphoreType.DMA,
        ],
        compiler_params=pltpu.CompilerParams(use_tc_tiling_on_sc=True),
    )
    def k(table_hbm, out_hbm, buf, sem):
        wid = lax.axis_index("s") * NC + lax.axis_index("c")
        stage = [pltpu.async_copy(table_hbm, buf.at[r], sem) for r in range(rep)]
        for c in stage:
            c.wait()
        copies = [
            pltpu.async_copy(
                buf, out_hbm.at[pl.ds((i * NW + wid) * rep, rep)], sem
            )
            for i in range(n_dma)
        ]
        for c in copies:
            c.wait()

    return k


def kernel(x, pos_table):
    B, L = x.shape
    D = pos_table.shape[1]
    info = plsc.get_sparse_core_info()
    NC, NS = info.num_cores, info.num_subcores
    k = _make_sc_broadcast(B, L, D, NC, NS)
    return k(pos_table[:L])


# speedup vs baseline: 2.0849x; 1.6342x over previous
import functools

import jax
import jax.numpy as jnp
from jax import lax
from jax.experimental import pallas as pl
from jax.experimental.pallas import tpu as pltpu
from jax.experimental.pallas import tpu_sc as plsc


def _make_sc_broadcast(B, L, D, NC, NS):
    NW = NC * NS
    rows_per_w = B // NW
    row_words = L * D
    rep = 1
    for cand in range(min(rows_per_w, (120 * 1024) // row_words), 0, -1):
        if rows_per_w % cand == 0 and cand * row_words * 4 <= 480 * 1024:
            rep = cand
            break
    n_dma = rows_per_w // rep

    mesh = plsc.VectorSubcoreMesh(core_axis_name="c", subcore_axis_name="s")

    @functools.partial(
        pl.kernel,
        mesh=mesh,
        out_type=jax.ShapeDtypeStruct((B, row_words), jnp.float32),
        scratch_types=[
            pltpu.VMEM((rep, row_words), jnp.float32),
            pltpu.SemaphoreType.DMA,
        ],
    )
    def k(table_hbm, out_hbm, buf, sem):
        wid = lax.axis_index("s") * NC + lax.axis_index("c")
        pltpu.sync_copy(table_hbm, buf.at[0])

        def replicate(i, carry):
            v = buf[0, pl.ds(i * 16, 16)]
            for r in range(1, rep):
                buf[r, pl.ds(i * 16, 16)] = v
            return carry

        lax.fori_loop(0, row_words // 16, replicate, 0)
        copies = [
            pltpu.async_copy(
                buf, out_hbm.at[pl.ds((i * NW + wid) * rep, rep)], sem
            )
            for i in range(n_dma)
        ]
        for c in copies:
            c.wait()

    return k


def kernel(x, pos_table):
    B, L = x.shape
    D = pos_table.shape[1]
    info = plsc.get_sparse_core_info()
    NC, NS = info.num_cores, info.num_subcores
    table_flat = pos_table[:L].reshape(L * D)
    out = _make_sc_broadcast(B, L, D, NC, NS)(table_flat)
    return out.reshape(B, L, D)
